# baseline (speedup 1.0000x reference)
import functools

import jax
import jax.numpy as jnp
import numpy as np
from jax import lax
from jax.experimental import pallas as pl
from jax.experimental.pallas import tpu as pltpu
from jax.experimental.pallas import tpu_sc as plsc

R = 64
N = 8192
NC = 2
NS = 16
L = 16
NW = NC * NS
CH = N // L
U = 8
OUTER = CH // U
K = 32
BR = 8


def _threefry2x32(k0, k1, x0, x1):
    rot1 = (13, 15, 26, 6)
    rot2 = (17, 29, 16, 24)
    ks0 = np.uint32(k0)
    ks1 = np.uint32(k1)
    ks2 = np.uint32(0x1BD11BDA) ^ ks0 ^ ks1
    x0 = (x0 + ks0).astype(np.uint32)
    x1 = (x1 + ks1).astype(np.uint32)

    def rotl(v, r):
        return ((v << np.uint32(r)) | (v >> np.uint32(32 - r))).astype(np.uint32)

    for rots, a0, a1, inc in ((rot1, ks1, ks2, 1), (rot2, ks2, ks0, 2),
                              (rot1, ks0, ks1, 3), (rot2, ks1, ks2, 4),
                              (rot1, ks2, ks0, 5)):
        for r in rots:
            x0 = (x0 + x1).astype(np.uint32)
            x1 = rotl(x1, r) ^ x0
        x0 = (x0 + a0).astype(np.uint32)
        x1 = (x1 + a1 + np.uint32(inc)).astype(np.uint32)
    return x0, x1


def _random_unit_floats(k0, k1, n):
    o0, o1 = _threefry2x32(k0, k1, np.zeros(n, np.uint32),
                           np.arange(n, dtype=np.uint32))
    bits = o0 ^ o1
    fb = ((bits >> np.uint32(9)) | np.uint32(0x3F800000)).view(np.float32)
    return fb - np.float32(1.0)


def _erfinv(x):
    x = x.astype(np.float64)
    w = -np.log1p(-x * x)
    ws = w - 2.5
    wl = np.sqrt(np.maximum(w, 5.0)) - 3.0
    ps = np.full_like(x, 2.81022636e-08)
    for cc in (3.43273939e-07, -3.5233877e-06, -4.39150654e-06, 0.00021858087,
               -0.00125372503, -0.00417768164, 0.246640727, 1.50140941):
        ps = cc + ps * ws
    pb = np.full_like(x, -0.000200214257)
    for cc in (0.000100950558, 0.00134934322, -0.00367342844, 0.00573950773,
               -0.0076224613, 0.00943887047, 1.00167406, 2.83297682):
        pb = cc + pb * wl
    return np.where(w < 5.0, ps, pb) * x


def _make_constants():
    b1, b2 = _threefry2x32(0, 42, np.zeros(2, np.uint32),
                           np.arange(2, dtype=np.uint32))
    fu = _random_unit_floats(b1[0], b2[0], R * N)
    u = np.maximum(np.float32(0.1),
                   fu * np.float32(0.2) + np.float32(0.1)).reshape(R, N)
    fn = _random_unit_floats(b1[1], b2[1], R * N)
    lo = np.nextafter(np.float32(-1.0), np.float32(0.0))
    un = np.maximum(lo, fn * (np.float32(1.0) - lo) + lo)
    noise = (np.sqrt(2.0) * _erfinv(un)).astype(np.float32).reshape(R, N)
    return u, noise


_A, _B = _make_constants()
_BP = _B - _B.max(axis=-1, keepdims=True)


def _row_compute(xv, av, bv, ev, lanes):
    def p1(i, carry):
        ibc = jnp.zeros((L,), jnp.int32) + i
        out = []
        for j in range(U):
            bm, bi = carry[j]
            x16 = xv[pl.ds(i * (U * L) + j * L, L)]
            take = x16 > bm
            out.append((jnp.where(take, x16, bm), jnp.where(take, ibc, bi)))
        return tuple(out)

    init = tuple((jnp.full((L,), -jnp.inf, jnp.float32),
                  jnp.zeros((L,), jnp.int32)) for _ in range(U))
    accs = lax.fori_loop(0, OUTER, p1, init)

    merged = None
    for j in range(U):
        bm, bi = accs[j]
        g = bi * (U * L) + (lanes + j * L)
        if merged is None:
            merged = (bm, g)
        else:
            pm, pg = merged
            take = (bm > pm) | ((bm == pm) & (g < pg))
            merged = (jnp.where(take, bm, pm), jnp.where(take, g, pg))
    bm, bg = merged
    m = bm[0]
    midx = bg[0]
    for l in range(1, L):
        v = bm[l]
        gi = bg[l]
        take = (v > m) | ((v == m) & (gi < midx))
        m = jnp.where(take, v, m)
        midx = jnp.where(take, gi, midx)

    def p2(i, sacc):
        out = list(sacc)
        for j in range(U):
            off = i * (U * L) + j * L
            a16 = av[pl.ds(off, L)]
            b16 = bv[pl.ds(off, L)]
            e16 = jnp.exp(m * a16 + b16)
            ev[pl.ds(off, L)] = e16
            out[j] = out[j] + e16
        return tuple(out)

    sinit = tuple(jnp.zeros((L,), jnp.float32) for _ in range(U))
    saccs = lax.fori_loop(0, OUTER, p2, sinit)
    svec = saccs[0]
    for j in range(1, U):
        svec = svec + saccs[j]

    coff = (midx >> 4) * L
    bfix = bv[pl.ds(coff, L)]
    eold = ev[pl.ds(coff, L)]
    sel = lanes == (midx & (L - 1))
    efix = jnp.where(sel, jnp.exp(m + bfix), eold)
    ev[pl.ds(coff, L)] = efix
    svec = svec + (efix - eold)

    s = svec[0]
    for l in range(1, L):
        s = s + svec[l]
    rinv = jnp.full((L,), jnp.float32(1.0)) / (jnp.zeros((L,), jnp.float32) + s)

    def p3(i, carry):
        for j in range(U):
            off = i * (U * L) + j * L
            ev[pl.ds(off, L)] = ev[pl.ds(off, L)] * rinv
        return carry

    lax.fori_loop(0, OUTER, p3, 0)


def _sc_body(x_hbm, a_hbm, b_hbm, out_hbm, xv, av, bv, ev, sx, sab, so):
    wid = lax.axis_index("s") * NC + lax.axis_index("c")
    lanes = lax.iota(jnp.int32, L)
    rows_per_w = K // NW
    outh = []
    for rr in range(rows_per_w):
        row = wid * rows_per_w + rr
        hx = pltpu.async_copy(x_hbm.at[row], xv, sx)
        ha = pltpu.async_copy(a_hbm.at[row], av, sab)
        hb = pltpu.async_copy(b_hbm.at[row], bv, sab)
        hx.wait()
        ha.wait()
        hb.wait()
        _row_compute(xv, av, bv, ev, lanes)
        outh.append(pltpu.async_copy(ev, out_hbm.at[row], so))
    for h in outh:
        h.wait()


_sc = functools.partial(
    pl.kernel,
    out_type=jax.ShapeDtypeStruct((K, N), jnp.float32),
    mesh=plsc.VectorSubcoreMesh(core_axis_name="c", subcore_axis_name="s"),
    scratch_types=(
        [pltpu.VMEM((N,), jnp.float32) for _ in range(4)]
        + [pltpu.SemaphoreType.DMA for _ in range(3)]
    ),
)(_sc_body)


def _tc_body(x_ref, a_ref, b_ref, o_ref):
    x = x_ref[...]
    a = a_ref[...]
    b = b_ref[...]
    m = jnp.max(x, axis=-1, keepdims=True)
    col = lax.broadcasted_iota(jnp.int32, (BR, N), 1)
    midx = jnp.min(jnp.where(x == m, col, jnp.int32(2**31 - 1)),
                   axis=-1, keepdims=True)
    t = jnp.where(col == midx, m, m * a) + b
    e = jnp.exp(t)
    o_ref[...] = e / jnp.sum(e, axis=-1, keepdims=True)


_tc = pl.pallas_call(
    _tc_body,
    out_shape=jax.ShapeDtypeStruct((R - K, N), jnp.float32),
    grid=((R - K) // BR,),
    in_specs=[pl.BlockSpec((BR, N), lambda i: (i, 0))] * 3,
    out_specs=pl.BlockSpec((BR, N), lambda i: (i, 0)),
)


def kernel(x):
    top = _sc(x[:K], _A[:K], _BP[:K])
    bot = _tc(x[K:], _A[K:], _BP[K:])
    return jnp.concatenate([top, bot], axis=0)


# speedup vs baseline: 1.0916x; 1.0916x over previous
import functools

import jax
import jax.numpy as jnp
import numpy as np
from jax import lax
from jax.experimental import pallas as pl
from jax.experimental.pallas import tpu as pltpu
from jax.experimental.pallas import tpu_sc as plsc

R = 64
N = 8192
NC = 2
NS = 16
L = 16
NW = NC * NS
CH = N // L
U = 8
OUTER = CH // U
K = 32
BR = 8


def _threefry2x32(k0, k1, x0, x1):
    rot1 = (13, 15, 26, 6)
    rot2 = (17, 29, 16, 24)
    ks0 = np.uint32(k0)
    ks1 = np.uint32(k1)
    ks2 = np.uint32(0x1BD11BDA) ^ ks0 ^ ks1
    x0 = (x0 + ks0).astype(np.uint32)
    x1 = (x1 + ks1).astype(np.uint32)

    def rotl(v, r):
        return ((v << np.uint32(r)) | (v >> np.uint32(32 - r))).astype(np.uint32)

    for rots, a0, a1, inc in ((rot1, ks1, ks2, 1), (rot2, ks2, ks0, 2),
                              (rot1, ks0, ks1, 3), (rot2, ks1, ks2, 4),
                              (rot1, ks2, ks0, 5)):
        for r in rots:
            x0 = (x0 + x1).astype(np.uint32)
            x1 = rotl(x1, r) ^ x0
        x0 = (x0 + a0).astype(np.uint32)
        x1 = (x1 + a1 + np.uint32(inc)).astype(np.uint32)
    return x0, x1


def _random_unit_floats(k0, k1, n):
    o0, o1 = _threefry2x32(k0, k1, np.zeros(n, np.uint32),
                           np.arange(n, dtype=np.uint32))
    bits = o0 ^ o1
    fb = ((bits >> np.uint32(9)) | np.uint32(0x3F800000)).view(np.float32)
    return fb - np.float32(1.0)


def _erfinv(x):
    x = x.astype(np.float64)
    w = -np.log1p(-x * x)
    ws = w - 2.5
    wl = np.sqrt(np.maximum(w, 5.0)) - 3.0
    ps = np.full_like(x, 2.81022636e-08)
    for cc in (3.43273939e-07, -3.5233877e-06, -4.39150654e-06, 0.00021858087,
               -0.00125372503, -0.00417768164, 0.246640727, 1.50140941):
        ps = cc + ps * ws
    pb = np.full_like(x, -0.000200214257)
    for cc in (0.000100950558, 0.00134934322, -0.00367342844, 0.00573950773,
               -0.0076224613, 0.00943887047, 1.00167406, 2.83297682):
        pb = cc + pb * wl
    return np.where(w < 5.0, ps, pb) * x


def _make_constants():
    b1, b2 = _threefry2x32(0, 42, np.zeros(2, np.uint32),
                           np.arange(2, dtype=np.uint32))
    fu = _random_unit_floats(b1[0], b2[0], R * N)
    u = np.maximum(np.float32(0.1),
                   fu * np.float32(0.2) + np.float32(0.1)).reshape(R, N)
    fn = _random_unit_floats(b1[1], b2[1], R * N)
    lo = np.nextafter(np.float32(-1.0), np.float32(0.0))
    un = np.maximum(lo, fn * (np.float32(1.0) - lo) + lo)
    noise = (np.sqrt(2.0) * _erfinv(un)).astype(np.float32).reshape(R, N)
    return u, noise


_A, _B = _make_constants()
_BP = _B - _B.max(axis=-1, keepdims=True)


def _row_compute(xv, av, bv, ev, lanes):
    def p1(i, carry):
        ibc = jnp.zeros((L,), jnp.int32) + i
        out = []
        for j in range(U):
            bm, bi = carry[j]
            x16 = xv[pl.ds(i * (U * L) + j * L, L)]
            take = x16 > bm
            out.append((jnp.where(take, x16, bm), jnp.where(take, ibc, bi)))
        return tuple(out)

    init = tuple((jnp.full((L,), -jnp.inf, jnp.float32),
                  jnp.zeros((L,), jnp.int32)) for _ in range(U))
    accs = lax.fori_loop(0, OUTER, p1, init)

    merged = None
    for j in range(U):
        bm, bi = accs[j]
        g = bi * (U * L) + (lanes + j * L)
        if merged is None:
            merged = (bm, g)
        else:
            pm, pg = merged
            take = (bm > pm) | ((bm == pm) & (g < pg))
            merged = (jnp.where(take, bm, pm), jnp.where(take, g, pg))
    bm, bg = merged
    m = bm[0]
    midx = bg[0]
    for l in range(1, L):
        v = bm[l]
        gi = bg[l]
        take = (v > m) | ((v == m) & (gi < midx))
        m = jnp.where(take, v, m)
        midx = jnp.where(take, gi, midx)

    def p2(i, sacc):
        out = list(sacc)
        for j in range(U):
            off = i * (U * L) + j * L
            a16 = av[pl.ds(off, L)]
            b16 = bv[pl.ds(off, L)]
            e16 = jnp.exp(m * a16 + b16)
            ev[pl.ds(off, L)] = e16
            out[j] = out[j] + e16
        return tuple(out)

    sinit = tuple(jnp.zeros((L,), jnp.float32) for _ in range(U))
    saccs = lax.fori_loop(0, OUTER, p2, sinit)
    svec = saccs[0]
    for j in range(1, U):
        svec = svec + saccs[j]

    coff = (midx >> 4) * L
    bfix = bv[pl.ds(coff, L)]
    eold = ev[pl.ds(coff, L)]
    sel = lanes == (midx & (L - 1))
    efix = jnp.where(sel, jnp.exp(m + bfix), eold)
    ev[pl.ds(coff, L)] = efix
    svec = svec + (efix - eold)

    s = svec[0]
    for l in range(1, L):
        s = s + svec[l]
    rinv = jnp.full((L,), jnp.float32(1.0)) / (jnp.zeros((L,), jnp.float32) + s)

    def p3(i, carry):
        for j in range(U):
            off = i * (U * L) + j * L
            ev[pl.ds(off, L)] = ev[pl.ds(off, L)] * rinv
        return carry

    lax.fori_loop(0, OUTER, p3, 0)


def _sc_body(x_hbm, a_hbm, b_hbm, out_hbm, xv, av, bv, ev, sx, sab, so):
    wid = lax.axis_index("s") * NC + lax.axis_index("c")
    lanes = lax.iota(jnp.int32, L)
    rows_per_w = K // NW
    outh = []
    for rr in range(rows_per_w):
        row = wid * rows_per_w + rr
        hx = pltpu.async_copy(x_hbm.at[row], xv, sx)
        ha = pltpu.async_copy(a_hbm.at[row], av, sab)
        hb = pltpu.async_copy(b_hbm.at[row], bv, sab)
        hx.wait()
        ha.wait()
        hb.wait()
        _row_compute(xv, av, bv, ev, lanes)
        outh.append(pltpu.async_copy(ev, out_hbm.at[row], so))
    for h in outh:
        h.wait()


_sc = functools.partial(
    pl.kernel,
    out_type=jax.ShapeDtypeStruct((K, N), jnp.float32),
    mesh=plsc.VectorSubcoreMesh(core_axis_name="c", subcore_axis_name="s"),
    scratch_types=(
        [pltpu.VMEM((N,), jnp.float32) for _ in range(4)]
        + [pltpu.SemaphoreType.DMA for _ in range(3)]
    ),
)(_sc_body)


def _tc_body(x_ref, a_ref, b_ref, o_ref):
    x = x_ref[...]
    a = a_ref[...]
    b = b_ref[...]
    m = jnp.max(x, axis=-1, keepdims=True)
    col = lax.broadcasted_iota(jnp.int32, (BR, N), 1)
    midx = jnp.min(jnp.where(x == m, col, jnp.int32(2**31 - 1)),
                   axis=-1, keepdims=True)
    t = jnp.where(col == midx, m, m * a) + b
    e = jnp.exp(t)
    o_ref[...] = e / jnp.sum(e, axis=-1, keepdims=True)


_KB = K // BR

_tc = pl.pallas_call(
    _tc_body,
    out_shape=jax.ShapeDtypeStruct((R, N), jnp.float32),
    grid=((R - K) // BR,),
    in_specs=[pl.BlockSpec((BR, N), lambda i: (i + _KB, 0))] * 3,
    out_specs=pl.BlockSpec((BR, N), lambda i: (i + _KB, 0)),
)


def kernel(x):
    top = _sc(x, _A, _BP)
    full = _tc(x, _A, _BP)
    return lax.dynamic_update_slice(full, top, (0, 0))


# speedup vs baseline: 5.0813x; 4.6551x over previous
import functools

import jax
import jax.numpy as jnp
import numpy as np
from jax import lax
from jax.experimental import pallas as pl
from jax.experimental.pallas import tpu as pltpu

R = 64
N = 8192
BR = 16


def _threefry2x32(k0, k1, x0, x1):
    rot1 = (13, 15, 26, 6)
    rot2 = (17, 29, 16, 24)
    ks0 = np.uint32(k0)
    ks1 = np.uint32(k1)
    ks2 = np.uint32(0x1BD11BDA) ^ ks0 ^ ks1
    x0 = (x0 + ks0).astype(np.uint32)
    x1 = (x1 + ks1).astype(np.uint32)

    def rotl(v, r):
        return ((v << np.uint32(r)) | (v >> np.uint32(32 - r))).astype(np.uint32)

    for rots, a0, a1, inc in ((rot1, ks1, ks2, 1), (rot2, ks2, ks0, 2),
                              (rot1, ks0, ks1, 3), (rot2, ks1, ks2, 4),
                              (rot1, ks2, ks0, 5)):
        for r in rots:
            x0 = (x0 + x1).astype(np.uint32)
            x1 = rotl(x1, r) ^ x0
        x0 = (x0 + a0).astype(np.uint32)
        x1 = (x1 + a1 + np.uint32(inc)).astype(np.uint32)
    return x0, x1


def _random_unit_floats(k0, k1, n):
    o0, o1 = _threefry2x32(k0, k1, np.zeros(n, np.uint32),
                           np.arange(n, dtype=np.uint32))
    bits = o0 ^ o1
    fb = ((bits >> np.uint32(9)) | np.uint32(0x3F800000)).view(np.float32)
    return fb - np.float32(1.0)


def _erfinv(x):
    x = x.astype(np.float64)
    w = -np.log1p(-x * x)
    ws = w - 2.5
    wl = np.sqrt(np.maximum(w, 5.0)) - 3.0
    ps = np.full_like(x, 2.81022636e-08)
    for cc in (3.43273939e-07, -3.5233877e-06, -4.39150654e-06, 0.00021858087,
               -0.00125372503, -0.00417768164, 0.246640727, 1.50140941):
        ps = cc + ps * ws
    pb = np.full_like(x, -0.000200214257)
    for cc in (0.000100950558, 0.00134934322, -0.00367342844, 0.00573950773,
               -0.0076224613, 0.00943887047, 1.00167406, 2.83297682):
        pb = cc + pb * wl
    return np.where(w < 5.0, ps, pb) * x


def _make_constants():
    b1, b2 = _threefry2x32(0, 42, np.zeros(2, np.uint32),
                           np.arange(2, dtype=np.uint32))
    fu = _random_unit_floats(b1[0], b2[0], R * N)
    u = np.maximum(np.float32(0.1),
                   fu * np.float32(0.2) + np.float32(0.1)).reshape(R, N)
    fn = _random_unit_floats(b1[1], b2[1], R * N)
    lo = np.nextafter(np.float32(-1.0), np.float32(0.0))
    un = np.maximum(lo, fn * (np.float32(1.0) - lo) + lo)
    noise = (np.sqrt(2.0) * _erfinv(un)).astype(np.float32).reshape(R, N)
    return u, noise


_A, _B = _make_constants()
_BP = _B - _B.max(axis=-1, keepdims=True)


def _tc_body(x_ref, a_ref, b_ref, o_ref):
    x = x_ref[...]
    a = a_ref[...]
    b = b_ref[...]
    m = jnp.max(x, axis=-1, keepdims=True)
    col = lax.broadcasted_iota(jnp.int32, (BR, N), 1)
    midx = jnp.min(jnp.where(x == m, col, jnp.int32(2**31 - 1)),
                   axis=-1, keepdims=True)
    t = jnp.where(col == midx, m, m * a) + b
    e = jnp.exp(t)
    o_ref[...] = e * (jnp.float32(1.0) / jnp.sum(e, axis=-1, keepdims=True))


_tc = pl.pallas_call(
    _tc_body,
    out_shape=jax.ShapeDtypeStruct((R, N), jnp.float32),
    grid=(R // BR,),
    in_specs=[pl.BlockSpec((BR, N), lambda i: (i, 0))] * 3,
    out_specs=pl.BlockSpec((BR, N), lambda i: (i, 0)),
)


def kernel(x):
    return _tc(x, _A, _BP)


# speedup vs baseline: 6.4794x; 1.2751x over previous
import functools

import jax
import jax.numpy as jnp
import numpy as np
from jax import lax
from jax.experimental import pallas as pl
from jax.experimental.pallas import tpu as pltpu

R = 64
N = 8192
BR = 32


def _threefry2x32(k0, k1, x0, x1):
    rot1 = (13, 15, 26, 6)
    rot2 = (17, 29, 16, 24)
    ks0 = np.uint32(k0)
    ks1 = np.uint32(k1)
    ks2 = np.uint32(0x1BD11BDA) ^ ks0 ^ ks1
    x0 = (x0 + ks0).astype(np.uint32)
    x1 = (x1 + ks1).astype(np.uint32)

    def rotl(v, r):
        return ((v << np.uint32(r)) | (v >> np.uint32(32 - r))).astype(np.uint32)

    for rots, a0, a1, inc in ((rot1, ks1, ks2, 1), (rot2, ks2, ks0, 2),
                              (rot1, ks0, ks1, 3), (rot2, ks1, ks2, 4),
                              (rot1, ks2, ks0, 5)):
        for r in rots:
            x0 = (x0 + x1).astype(np.uint32)
            x1 = rotl(x1, r) ^ x0
        x0 = (x0 + a0).astype(np.uint32)
        x1 = (x1 + a1 + np.uint32(inc)).astype(np.uint32)
    return x0, x1


def _random_unit_floats(k0, k1, n):
    o0, o1 = _threefry2x32(k0, k1, np.zeros(n, np.uint32),
                           np.arange(n, dtype=np.uint32))
    bits = o0 ^ o1
    fb = ((bits >> np.uint32(9)) | np.uint32(0x3F800000)).view(np.float32)
    return fb - np.float32(1.0)


def _erfinv(x):
    x = x.astype(np.float64)
    w = -np.log1p(-x * x)
    ws = w - 2.5
    wl = np.sqrt(np.maximum(w, 5.0)) - 3.0
    ps = np.full_like(x, 2.81022636e-08)
    for cc in (3.43273939e-07, -3.5233877e-06, -4.39150654e-06, 0.00021858087,
               -0.00125372503, -0.00417768164, 0.246640727, 1.50140941):
        ps = cc + ps * ws
    pb = np.full_like(x, -0.000200214257)
    for cc in (0.000100950558, 0.00134934322, -0.00367342844, 0.00573950773,
               -0.0076224613, 0.00943887047, 1.00167406, 2.83297682):
        pb = cc + pb * wl
    return np.where(w < 5.0, ps, pb) * x


def _make_constants():
    b1, b2 = _threefry2x32(0, 42, np.zeros(2, np.uint32),
                           np.arange(2, dtype=np.uint32))
    fu = _random_unit_floats(b1[0], b2[0], R * N)
    u = np.maximum(np.float32(0.1),
                   fu * np.float32(0.2) + np.float32(0.1)).reshape(R, N)
    fn = _random_unit_floats(b1[1], b2[1], R * N)
    lo = np.nextafter(np.float32(-1.0), np.float32(0.0))
    un = np.maximum(lo, fn * (np.float32(1.0) - lo) + lo)
    noise = (np.sqrt(2.0) * _erfinv(un)).astype(np.float32).reshape(R, N)
    return u, noise


_A, _B = _make_constants()
_BP = _B - _B.max(axis=-1, keepdims=True)


def _tc_body(x_ref, a_ref, b_ref, o_ref):
    x = x_ref[...]
    a = a_ref[...]
    b = b_ref[...]
    m = jnp.max(x, axis=-1, keepdims=True)
    col = lax.broadcasted_iota(jnp.int32, (BR, N), 1)
    midx = jnp.min(jnp.where(x == m, col, jnp.int32(2**31 - 1)),
                   axis=-1, keepdims=True)
    t = jnp.where(col == midx, m, m * a) + b
    e = jnp.exp(t)
    o_ref[...] = e * (jnp.float32(1.0) / jnp.sum(e, axis=-1, keepdims=True))


_tc = pl.pallas_call(
    _tc_body,
    out_shape=jax.ShapeDtypeStruct((R, N), jnp.float32),
    grid=(R // BR,),
    in_specs=[pl.BlockSpec((BR, N), lambda i: (i, 0))] * 3,
    out_specs=pl.BlockSpec((BR, N), lambda i: (i, 0)),
)


def kernel(x):
    return _tc(x, _A, _BP)
